# baseline (device time: 202896 ns/iter reference)
import jax
import jax.numpy as jnp
from jax import lax
from jax.experimental import pallas as pl
from jax.experimental.pallas import tpu as pltpu

N_DEV = 8
S = 512
D = 1024
H = 8
DH = 128
SCALE = 0.08838834764831843


def kernel(x, Wq, Wo, Wk, Wv):
    xb = x.reshape(S, D).astype(jnp.bfloat16)
    wq = Wq.astype(jnp.bfloat16)
    wk = Wk.astype(jnp.bfloat16)
    wv = Wv.astype(jnp.bfloat16)
    wo = Wo.astype(jnp.bfloat16)

    def body(x_ref, wq_ref, wo_ref, wk_ref, wv_ref, out_ref,
             xrecv, precv, pstage,
             xsend_sems, xrecv_sems, psend_sems, precv_sems):
        my = lax.axis_index("i")

        bar = pltpu.get_barrier_semaphore()
        for k in range(1, N_DEV):
            pl.semaphore_signal(
                bar, inc=1,
                device_id=((my + k) % N_DEV,),
                device_id_type=pl.DeviceIdType.MESH,
            )
        pl.semaphore_wait(bar, N_DEV - 1)

        xrecv[my, :, :] = x_ref[:, :]

        x_rdmas = []
        for k in range(1, N_DEV):
            tgt = (my + k) % N_DEV
            r = pltpu.make_async_remote_copy(
                src_ref=xrecv.at[my],
                dst_ref=xrecv.at[my],
                send_sem=xsend_sems.at[k],
                recv_sem=xrecv_sems.at[my],
                device_id=(tgt,),
                device_id_type=pl.DeviceIdType.MESH,
            )
            r.start()
            x_rdmas.append(r)

        def compute_partial(xg):
            q = jnp.dot(xg, wq_ref[:, :], preferred_element_type=jnp.float32)
            kk = jnp.dot(xg, wk_ref[:, :], preferred_element_type=jnp.float32)
            v = jnp.dot(xg, wv_ref[:, :], preferred_element_type=jnp.float32)
            outs = []
            for h in range(H):
                sl = slice(h * DH, (h + 1) * DH)
                qh = q[:, sl].astype(jnp.bfloat16)
                kh = kk[:, sl].astype(jnp.bfloat16)
                vh = v[:, sl].astype(jnp.bfloat16)
                s = lax.dot_general(
                    qh, kh, (((1,), (1,)), ((), ())),
                    preferred_element_type=jnp.float32,
                ) * SCALE
                m = jnp.max(s, axis=1, keepdims=True)
                p = jnp.exp(s - m)
                l = jnp.sum(p, axis=1, keepdims=True)
                oh = jnp.dot(
                    p.astype(jnp.bfloat16), vh,
                    preferred_element_type=jnp.float32,
                )
                outs.append(oh / l)
            ao = jnp.concatenate(outs, axis=1).astype(jnp.bfloat16)
            return jnp.dot(ao, wo_ref[:, :], preferred_element_type=jnp.float32)

        out_ref[0, :, :] = compute_partial(xrecv[my])

        p_rdmas = {}
        for k in range(1, N_DEV):
            g = (my + k) % N_DEV
            pltpu.make_async_remote_copy(
                src_ref=xrecv.at[g],
                dst_ref=xrecv.at[g],
                send_sem=xsend_sems.at[k],
                recv_sem=xrecv_sems.at[g],
                device_id=(my,),
                device_id_type=pl.DeviceIdType.MESH,
            ).wait_recv()

            slot = k % 2
            if k >= 3:
                p_rdmas[k - 2].wait_send()
            pstage[slot, :, :] = compute_partial(xrecv[g]).astype(jnp.bfloat16)
            r = pltpu.make_async_remote_copy(
                src_ref=pstage.at[slot],
                dst_ref=precv.at[my],
                send_sem=psend_sems.at[slot],
                recv_sem=precv_sems.at[my],
                device_id=(g,),
                device_id_type=pl.DeviceIdType.MESH,
            )
            r.start()
            p_rdmas[k] = r

        for k in range(1, N_DEV):
            g = (my + k) % N_DEV
            pltpu.make_async_remote_copy(
                src_ref=precv.at[g],
                dst_ref=precv.at[g],
                send_sem=psend_sems.at[0],
                recv_sem=precv_sems.at[g],
                device_id=(my,),
                device_id_type=pl.DeviceIdType.MESH,
            ).wait_recv()
            out_ref[0, :, :] = out_ref[0, :, :] + precv[g].astype(jnp.float32)

        for r in x_rdmas:
            r.wait_send()
        for k in (N_DEV - 2, N_DEV - 1):
            p_rdmas[k].wait_send()

    return pl.pallas_call(
        body,
        out_shape=jax.ShapeDtypeStruct((1, S, D), jnp.float32),
        in_specs=[pl.BlockSpec(memory_space=pltpu.VMEM)] * 5,
        out_specs=pl.BlockSpec(memory_space=pltpu.VMEM),
        scratch_shapes=[
            pltpu.VMEM((N_DEV, S, D), jnp.bfloat16),
            pltpu.VMEM((N_DEV, S, D), jnp.bfloat16),
            pltpu.VMEM((2, S, D), jnp.bfloat16),
            pltpu.SemaphoreType.DMA((N_DEV,)),
            pltpu.SemaphoreType.DMA((N_DEV,)),
            pltpu.SemaphoreType.DMA((2,)),
            pltpu.SemaphoreType.DMA((N_DEV,)),
        ],
        compiler_params=pltpu.CompilerParams(collective_id=0),
    )(xb, wq, wo, wk, wv)


# device time: 125277 ns/iter; 1.6196x vs baseline; 1.6196x over previous
import os

import jax
import jax.numpy as jnp
from jax import lax
from jax.experimental import pallas as pl
from jax.experimental.pallas import tpu as pltpu

_VARIANT = os.environ.get("KERNEL_VARIANT", "")

N_DEV = 8
S = 512
D = 1024
H = 8
DH = 128
SCALE = 0.08838834764831843


def _compute_partial(xg, wq_ref, wk_ref, wv_ref, wo_ref):
    q = jnp.dot(xg, wq_ref[:, :], preferred_element_type=jnp.float32)
    kk = jnp.dot(xg, wk_ref[:, :], preferred_element_type=jnp.float32)
    v = jnp.dot(xg, wv_ref[:, :], preferred_element_type=jnp.float32)
    outs = []
    for h in range(H):
        sl = slice(h * DH, (h + 1) * DH)
        qh = q[:, sl].astype(jnp.bfloat16)
        kh = kk[:, sl].astype(jnp.bfloat16)
        vh = v[:, sl].astype(jnp.bfloat16)
        s = lax.dot_general(
            qh, kh, (((1,), (1,)), ((), ())),
            preferred_element_type=jnp.float32,
        ) * SCALE
        m = jnp.max(s, axis=1, keepdims=True)
        p = jnp.exp(s - m)
        l = jnp.sum(p, axis=1, keepdims=True)
        oh = jnp.dot(
            p.astype(jnp.bfloat16), vh,
            preferred_element_type=jnp.float32,
        )
        outs.append(oh / l)
    ao = jnp.concatenate(outs, axis=1).astype(jnp.bfloat16)
    return jnp.dot(ao, wo_ref[:, :], preferred_element_type=jnp.float32)


def kernel(x, Wq, Wo, Wk, Wv):
    xb = x.reshape(S, D).astype(jnp.bfloat16)
    wq = Wq.astype(jnp.bfloat16)
    wk = Wk.astype(jnp.bfloat16)
    wv = Wv.astype(jnp.bfloat16)
    wo = Wo.astype(jnp.bfloat16)

    def body(x_ref, wq_ref, wo_ref, wk_ref, wv_ref, out_ref,
             xrecv, precv, pstage, pall,
             xsend_sems, xrecv_sems, psend_sems, precv_sems):
        my = lax.axis_index("i")

        if _VARIANT == "compute_only":
            xrecv[0, :, :] = x_ref[:, :]
            out_ref[0, :, :] = _compute_partial(
                xrecv[0], wq_ref, wk_ref, wv_ref, wo_ref)
            for k in range(1, N_DEV):
                out_ref[0, :, :] = out_ref[0, :, :] + _compute_partial(
                    xrecv[k], wq_ref, wk_ref, wv_ref, wo_ref)
            return

        bar = pltpu.get_barrier_semaphore()
        for k in range(1, N_DEV):
            pl.semaphore_signal(
                bar, inc=1,
                device_id=((my + k) % N_DEV,),
                device_id_type=pl.DeviceIdType.MESH,
            )
        pl.semaphore_wait(bar, N_DEV - 1)

        xrecv[my, :, :] = x_ref[:, :]

        x_rdmas = []
        for i, mask in enumerate((1, 3, 4)):
            r = pltpu.make_async_remote_copy(
                src_ref=xrecv.at[my],
                dst_ref=xrecv.at[my],
                send_sem=xsend_sems.at[i + 1],
                recv_sem=xrecv_sems.at[my],
                device_id=(my ^ mask,),
                device_id_type=pl.DeviceIdType.MESH,
            )
            r.start()
            x_rdmas.append(r)

        def compute_partial(xg):
            return _compute_partial(xg, wq_ref, wk_ref, wv_ref, wo_ref)

        out_ref[0, :, :] = compute_partial(xrecv[my])


        def wait_precv(slot):
            pltpu.make_async_remote_copy(
                src_ref=precv.at[slot],
                dst_ref=precv.at[slot],
                send_sem=psend_sems.at[0],
                recv_sem=precv_sems.at[slot],
                device_id=(my,),
                device_id_type=pl.DeviceIdType.MESH,
            ).wait_recv()

        def send_reduce(src, tgt_mask, dst_slot, sem):
            r = pltpu.make_async_remote_copy(
                src_ref=src,
                dst_ref=precv.at[dst_slot],
                send_sem=psend_sems.at[sem],
                recv_sem=precv_sems.at[dst_slot],
                device_id=(my ^ tgt_mask,),
                device_id_type=pl.DeviceIdType.MESH,
            )
            r.start()
            return r

        p_rdmas = []
        plan = [
            (1, 3, 4, ("defer", 0)),
            (3, 4, 5, ("defer", 1)),
            (4, 1, 6, ("defer", 2)),
            (7, 1, 7, ("defer", 3)),
            (2, None, None, ("send", 0, 3, 5, 2)),
            (5, None, None, ("send", 1, 1, 4, 5)),
            (6, None, None, ("send", 2, 1, 3, 6)),
        ]
        for mask, relay_mask, ssem, action in plan:
            g = my ^ mask
            pltpu.make_async_remote_copy(
                src_ref=xrecv.at[g],
                dst_ref=xrecv.at[g],
                send_sem=xsend_sems.at[0],
                recv_sem=xrecv_sems.at[g],
                device_id=(my,),
                device_id_type=pl.DeviceIdType.MESH,
            ).wait_recv()

            if relay_mask is not None:
                r = pltpu.make_async_remote_copy(
                    src_ref=xrecv.at[g],
                    dst_ref=xrecv.at[g],
                    send_sem=xsend_sems.at[ssem],
                    recv_sem=xrecv_sems.at[g],
                    device_id=(my ^ relay_mask,),
                    device_id_type=pl.DeviceIdType.MESH,
                )
                r.start()
                x_rdmas.append(r)

            part = compute_partial(xrecv[g]).astype(jnp.bfloat16)
            if action[0] == "defer":
                pall[action[1], :, :] = part
            else:
                _, pslot, tgt_mask, dst_slot, psem = action
                pstage[pslot, :, :] = part
                p_rdmas.append(
                    send_reduce(pstage.at[pslot], tgt_mask, dst_slot, psem))

        for in_slot, pall_slot, tgt_mask, dst_slot, psem in (
            (5, 0, 1, 0, 1),
            (4, 2, 4, 2, 4),
            (3, 3, 4, 6, 7),
            (6, 1, 3, 1, 3),
        ):
            wait_precv(in_slot)
            precv[in_slot, :, :] = (
                precv[in_slot].astype(jnp.float32)
                + pall[pall_slot].astype(jnp.float32)
            ).astype(jnp.bfloat16)
            p_rdmas.append(
                send_reduce(precv.at[in_slot], tgt_mask, dst_slot, psem))

        for in_slot in (0, 2, 1):
            wait_precv(in_slot)
            out_ref[0, :, :] = out_ref[0, :, :] + precv[in_slot].astype(
                jnp.float32)

        for r in x_rdmas:
            r.wait_send()
        for r in p_rdmas:
            r.wait_send()

    return pl.pallas_call(
        body,
        out_shape=jax.ShapeDtypeStruct((1, S, D), jnp.float32),
        in_specs=[pl.BlockSpec(memory_space=pltpu.VMEM)] * 5,
        out_specs=pl.BlockSpec(memory_space=pltpu.VMEM),
        scratch_shapes=[
            pltpu.VMEM((N_DEV, S, D), jnp.bfloat16),
            pltpu.VMEM((7, S, D), jnp.bfloat16),
            pltpu.VMEM((3, S, D), jnp.bfloat16),
            pltpu.VMEM((4, S, D), jnp.bfloat16),
            pltpu.SemaphoreType.DMA((N_DEV,)),
            pltpu.SemaphoreType.DMA((N_DEV,)),
            pltpu.SemaphoreType.DMA((N_DEV,)),
            pltpu.SemaphoreType.DMA((7,)),
        ],
        compiler_params=pltpu.CompilerParams(
            collective_id=None if _VARIANT == "compute_only" else 0
        ),
    )(xb, wq, wo, wk, wv)


# device time: 111258 ns/iter; 1.8237x vs baseline; 1.1260x over previous
import os

import jax
import jax.numpy as jnp
from jax import lax
from jax.experimental import pallas as pl
from jax.experimental.pallas import tpu as pltpu

_VARIANT = os.environ.get("KERNEL_VARIANT", "")

N_DEV = 8
S = 512
D = 1024
H = 8
DH = 128
SCALE = 0.08838834764831843


def _compute_partial(xg, wqkv_ref, wo_ref):
    q = jnp.dot(xg, wqkv_ref[:, :D],
                preferred_element_type=jnp.float32).astype(jnp.bfloat16)
    kk = jnp.dot(xg, wqkv_ref[:, D:2 * D],
                 preferred_element_type=jnp.float32).astype(jnp.bfloat16)
    v = jnp.dot(xg, wqkv_ref[:, 2 * D:],
                preferred_element_type=jnp.float32).astype(jnp.bfloat16)
    outs = []
    for h in range(H):
        sl = slice(h * DH, (h + 1) * DH)
        qh = q[:, sl]
        kh = kk[:, sl]
        vh = v[:, sl]
        s = lax.dot_general(
            qh, kh, (((1,), (1,)), ((), ())),
            preferred_element_type=jnp.float32,
        ) * SCALE
        m = jnp.max(s, axis=1, keepdims=True)
        p = jnp.exp(s - m)
        l = jnp.sum(p, axis=1, keepdims=True)
        oh = jnp.dot(
            p.astype(jnp.bfloat16), vh,
            preferred_element_type=jnp.float32,
        )
        outs.append(oh / l)
    ao = jnp.concatenate(outs, axis=1).astype(jnp.bfloat16)
    return jnp.dot(ao, wo_ref[:, :], preferred_element_type=jnp.float32)


def kernel(x, Wq, Wo, Wk, Wv):
    xb = x.reshape(S, D).astype(jnp.bfloat16)
    wqkv = jnp.concatenate([Wq, Wk, Wv], axis=1).astype(jnp.bfloat16)
    wo = Wo.astype(jnp.bfloat16)

    def body(x_ref, wqkv_ref, wo_ref, out_ref,
             xrecv, precv, pstage, pall,
             xsend_sems, xrecv_sems, psend_sems, precv_sems):
        my = lax.axis_index("i")

        if _VARIANT == "compute_only":
            xrecv[0, :, :] = x_ref[:, :]
            out_ref[0, :, :] = _compute_partial(
                xrecv[0], wqkv_ref, wo_ref)
            for k in range(1, N_DEV):
                out_ref[0, :, :] = out_ref[0, :, :] + _compute_partial(
                    xrecv[k], wqkv_ref, wo_ref)
            return

        bar = pltpu.get_barrier_semaphore()
        for k in range(1, N_DEV):
            pl.semaphore_signal(
                bar, inc=1,
                device_id=((my + k) % N_DEV,),
                device_id_type=pl.DeviceIdType.MESH,
            )
        pl.semaphore_wait(bar, N_DEV - 1)

        xrecv[my, :, :] = x_ref[:, :]

        x_rdmas = []
        for i, mask in enumerate((1, 3, 4)):
            r = pltpu.make_async_remote_copy(
                src_ref=xrecv.at[my],
                dst_ref=xrecv.at[my],
                send_sem=xsend_sems.at[i + 1],
                recv_sem=xrecv_sems.at[my],
                device_id=(my ^ mask,),
                device_id_type=pl.DeviceIdType.MESH,
            )
            r.start()
            x_rdmas.append(r)

        def compute_partial(xg):
            return _compute_partial(xg, wqkv_ref, wo_ref)

        out_ref[0, :, :] = compute_partial(xrecv[my])


        def wait_precv(slot):
            pltpu.make_async_remote_copy(
                src_ref=precv.at[slot],
                dst_ref=precv.at[slot],
                send_sem=psend_sems.at[0],
                recv_sem=precv_sems.at[slot],
                device_id=(my,),
                device_id_type=pl.DeviceIdType.MESH,
            ).wait_recv()

        def send_reduce(src, tgt_mask, dst_slot, sem):
            r = pltpu.make_async_remote_copy(
                src_ref=src,
                dst_ref=precv.at[dst_slot],
                send_sem=psend_sems.at[sem],
                recv_sem=precv_sems.at[dst_slot],
                device_id=(my ^ tgt_mask,),
                device_id_type=pl.DeviceIdType.MESH,
            )
            r.start()
            return r

        p_rdmas = []
        plan = [
            (1, 3, 4, ("defer", 0)),
            (3, 4, 5, ("defer", 1)),
            (4, 1, 6, ("defer", 2)),
            (7, 1, 7, ("send", 3, 4, 6, 7)),
            (2, None, None, ("send", 0, 3, 5, 2)),
            (5, None, None, ("send", 1, 1, 4, 5)),
            (6, None, None, ("send", 2, 6, 3, 6)),
        ]
        for mask, relay_mask, ssem, action in plan:
            g = my ^ mask
            pltpu.make_async_remote_copy(
                src_ref=xrecv.at[g],
                dst_ref=xrecv.at[g],
                send_sem=xsend_sems.at[0],
                recv_sem=xrecv_sems.at[g],
                device_id=(my,),
                device_id_type=pl.DeviceIdType.MESH,
            ).wait_recv()

            if relay_mask is not None:
                r = pltpu.make_async_remote_copy(
                    src_ref=xrecv.at[g],
                    dst_ref=xrecv.at[g],
                    send_sem=xsend_sems.at[ssem],
                    recv_sem=xrecv_sems.at[g],
                    device_id=(my ^ relay_mask,),
                    device_id_type=pl.DeviceIdType.MESH,
                )
                r.start()
                x_rdmas.append(r)

            part = compute_partial(xrecv[g]).astype(jnp.bfloat16)
            if action[0] == "defer":
                pall[action[1], :, :] = part
            else:
                _, pslot, tgt_mask, dst_slot, psem = action
                stage = pall if pslot == 3 else pstage
                stage[pslot, :, :] = part
                p_rdmas.append(
                    send_reduce(stage.at[pslot], tgt_mask, dst_slot, psem))

        for in_slot, pall_slot, tgt_mask, dst_slot, psem in (
            (6, 1, 3, 1, 3),
            (5, 0, 1, 0, 1),
            (4, 2, 4, 2, 4),
        ):
            wait_precv(in_slot)
            precv[in_slot, :, :] = (
                precv[in_slot].astype(jnp.float32)
                + pall[pall_slot].astype(jnp.float32)
            ).astype(jnp.bfloat16)
            p_rdmas.append(
                send_reduce(precv.at[in_slot], tgt_mask, dst_slot, psem))

        for in_slot in (1, 0, 3, 2):
            wait_precv(in_slot)
            out_ref[0, :, :] = out_ref[0, :, :] + precv[in_slot].astype(
                jnp.float32)

        for r in x_rdmas:
            r.wait_send()
        for r in p_rdmas:
            r.wait_send()

    return pl.pallas_call(
        body,
        out_shape=jax.ShapeDtypeStruct((1, S, D), jnp.float32),
        in_specs=[pl.BlockSpec(memory_space=pltpu.VMEM)] * 3,
        out_specs=pl.BlockSpec(memory_space=pltpu.VMEM),
        scratch_shapes=[
            pltpu.VMEM((N_DEV, S, D), jnp.bfloat16),
            pltpu.VMEM((7, S, D), jnp.bfloat16),
            pltpu.VMEM((3, S, D), jnp.bfloat16),
            pltpu.VMEM((4, S, D), jnp.bfloat16),
            pltpu.SemaphoreType.DMA((N_DEV,)),
            pltpu.SemaphoreType.DMA((N_DEV,)),
            pltpu.SemaphoreType.DMA((N_DEV,)),
            pltpu.SemaphoreType.DMA((7,)),
        ],
        compiler_params=pltpu.CompilerParams(
            collective_id=None if _VARIANT == "compute_only" else 0
        ),
    )(xb, wqkv, wo)
